# baseline (speedup 1.0000x reference)
import jax
import jax.numpy as jnp
from jax.experimental import pallas as pl
from jax.experimental.pallas import tpu as pltpu

DIM = 4096
N_EXPERTS = 64
TOPK = 8
N_GROUPS = 8
GROUP_SIZE = N_EXPERTS // N_GROUPS
TOPK_GROUPS = 4
ROUTE_SCALE = 2.5
N_TOK = 8192

BLOCK_T = 1024
BLOCK_K = 1024
NEG = -1e30


def _routing(lt, b, w_out_ref, idx_out_ref):
    B = lt.shape[1]
    origT = jax.nn.sigmoid(lt)
    sT = origT + b

    g3 = sT.reshape(N_GROUPS, GROUP_SIZE, B)
    rid = jax.lax.broadcasted_iota(jnp.int32, (N_GROUPS, GROUP_SIZE, B), 1)
    m1 = jnp.max(g3, axis=1, keepdims=True)
    a1 = jnp.min(jnp.where(g3 == m1, rid, GROUP_SIZE), axis=1, keepdims=True)
    m2 = jnp.max(jnp.where(rid == a1, NEG, g3), axis=1, keepdims=True)
    gsc = (m1 + m2).reshape(N_GROUPS, B)

    grow = jax.lax.broadcasted_iota(jnp.int32, (N_GROUPS, B), 0)
    gsel = jnp.zeros((N_GROUPS, B), jnp.bool_)
    for _ in range(TOPK_GROUPS):
        m = jnp.max(gsc, axis=0, keepdims=True)
        a = jnp.min(jnp.where(gsc == m, grow, N_GROUPS), axis=0, keepdims=True)
        hit = grow == a
        gsel = gsel | hit
        gsc = jnp.where(hit, NEG, gsc)

    row_sel = jnp.broadcast_to(
        gsel.reshape(N_GROUPS, 1, B), (N_GROUPS, GROUP_SIZE, B)
    ).reshape(N_EXPERTS, B)
    v = jnp.where(row_sel, sT, NEG)

    rows = jax.lax.broadcasted_iota(jnp.int32, (N_EXPERTS, B), 0)
    idx_rows = []
    w_rows = []
    for _ in range(TOPK):
        m = jnp.max(v, axis=0, keepdims=True)
        a = jnp.min(jnp.where(v == m, rows, N_EXPERTS), axis=0, keepdims=True)
        hit = rows == a
        idx_rows.append(a)
        w_rows.append(jnp.sum(jnp.where(hit, origT, 0.0), axis=0, keepdims=True))
        v = jnp.where(hit, NEG, v)
    idxT = jnp.concatenate(idx_rows, axis=0)
    wT_r = jnp.concatenate(w_rows, axis=0)
    wT_r = (wT_r / jnp.sum(wT_r, axis=0, keepdims=True)) * ROUTE_SCALE

    w_out_ref[...] = wT_r.T
    idx_out_ref[...] = idxT.T


def _gate_block(x_ref, wT_ref, b_ref, w_out_ref, idx_out_ref, acc_ref):
    k = pl.program_id(1)
    nk = pl.num_programs(1)

    @pl.when(k == 0)
    def _init():
        acc_ref[...] = jnp.zeros_like(acc_ref)

    acc_ref[...] += jnp.dot(
        x_ref[...], wT_ref[...], preferred_element_type=jnp.float32
    )

    @pl.when(k == nk - 1)
    def _finish():
        _routing(acc_ref[...].T, b_ref[...], w_out_ref, idx_out_ref)


def kernel(x, weight, bias):
    n = x.shape[0]
    wT = weight.T
    b2 = bias.reshape(N_EXPERTS, 1)
    grid = (n // BLOCK_T, DIM // BLOCK_K)
    w_out, idx_out = pl.pallas_call(
        _gate_block,
        grid=grid,
        in_specs=[
            pl.BlockSpec((BLOCK_T, BLOCK_K), lambda i, k: (i, k)),
            pl.BlockSpec((BLOCK_K, N_EXPERTS), lambda i, k: (k, 0)),
            pl.BlockSpec((N_EXPERTS, 1), lambda i, k: (0, 0)),
        ],
        out_specs=[
            pl.BlockSpec((BLOCK_T, TOPK), lambda i, k: (i, 0)),
            pl.BlockSpec((BLOCK_T, TOPK), lambda i, k: (i, 0)),
        ],
        out_shape=[
            jax.ShapeDtypeStruct((n, TOPK), jnp.float32),
            jax.ShapeDtypeStruct((n, TOPK), jnp.int32),
        ],
        scratch_shapes=[pltpu.VMEM((BLOCK_T, N_EXPERTS), jnp.float32)],
        compiler_params=pltpu.CompilerParams(
            dimension_semantics=("arbitrary", "arbitrary"),
        ),
    )(x, wT, b2)
    return w_out, idx_out
